# baseline (speedup 1.0000x reference)
import functools
import jax
import jax.numpy as jnp
from jax import lax
from jax.experimental import pallas as pl
from jax.experimental.pallas import tpu as pltpu
from jax.experimental.pallas import tpu_sc as plsc

N = 10000
E = 640000
D_IN = 128
EDGE_DIM = 16
HEADS = 4
HID = 16
NODE_DIM = HEADS * HID
L = 4
G = 64


def _silu(x):
    return x * jax.nn.sigmoid(x)


def _ln(x, g, b):
    mu = jnp.mean(x, axis=-1, keepdims=True)
    var = jnp.var(x, axis=-1, keepdims=True)
    return (x - mu) / jnp.sqrt(var + 1e-5) * g + b


SC_NC = 2
SC_NS = 16
SC_NW = SC_NC * SC_NS
EW = E // SC_NW
CH = 80
NCH = EW // CH

_SC_MESH = plsc.VectorSubcoreMesh(
    core_axis_name="c", subcore_axis_name="s",
    num_cores=SC_NC, num_subcores=SC_NS)


def _wid():
    return lax.axis_index("s") * SC_NC + lax.axis_index("c")


def _edge_sum_body(u_hbm, v_hbm, w_hbm, src_hbm, dst_hbm, out_hbm,
                   sidx, didx, u_b, v_b, w_b, ov_b,
                   sem_g0, sem_g1, sem_o0, sem_o1):
    wid = _wid()
    base = wid * EW
    sems = (sem_g0, sem_g1)
    sem_ob = (sem_o0, sem_o1)
    pltpu.sync_copy(src_hbm.at[wid], sidx)
    pltpu.sync_copy(dst_hbm.at[wid], didx)

    def issue_loads(b, c):
        pltpu.async_copy(u_hbm.at[sidx.at[c]], u_b.at[b], sems[b])
        pltpu.async_copy(v_hbm.at[didx.at[c]], v_b.at[b], sems[b])
        pltpu.async_copy(w_hbm.at[pl.ds(base + c * CH, CH)],
                         w_b.at[b], sems[b])

    def wait_loads(b, c):
        pltpu.make_async_copy(u_hbm.at[sidx.at[c]], u_b.at[b],
                              sems[b]).wait()
        pltpu.make_async_copy(v_hbm.at[didx.at[c]], v_b.at[b],
                              sems[b]).wait()
        pltpu.make_async_copy(w_hbm.at[pl.ds(base + c * CH, CH)],
                              w_b.at[b], sems[b]).wait()

    def wait_out(b, c):
        pltpu.make_async_copy(ov_b.at[b],
                              out_hbm.at[pl.ds(base + c * CH, CH)],
                              sem_ob[b]).wait()

    def compute(b, c):
        for r in range(CH):
            ov_b[b, r, :] = u_b[b, r, :] + v_b[b, r, :] + w_b[b, r, :]
        pltpu.async_copy(ov_b.at[b], out_hbm.at[pl.ds(base + c * CH, CH)],
                         sem_ob[b])

    issue_loads(0, 0)

    @pl.loop(0, NCH, step=2)
    def _chunk(c):
        @pl.when(c > 0)
        def _():
            wait_out(1, c - 1)
        issue_loads(1, c + 1)
        wait_loads(0, c)

        @pl.when(c > 0)
        def _():
            wait_out(0, c - 2)
        compute(0, c)

        @pl.when(c + 2 < NCH)
        def _():
            issue_loads(0, c + 2)
        wait_loads(1, c + 1)
        compute(1, c + 1)

    wait_out(0, NCH - 2)
    wait_out(1, NCH - 1)


def _edge_sum_sc(u, v, w, src3, dst3):
    return pl.kernel(
        _edge_sum_body,
        out_type=jax.ShapeDtypeStruct((E, EDGE_DIM), jnp.float32),
        mesh=_SC_MESH,
        compiler_params=pltpu.CompilerParams(use_tc_tiling_on_sc=False, needs_layout_passes=False),
        scratch_types=[
            pltpu.VMEM((NCH, CH), jnp.int32),
            pltpu.VMEM((NCH, CH), jnp.int32),
            pltpu.VMEM((2, CH, EDGE_DIM), jnp.float32),
            pltpu.VMEM((2, CH, EDGE_DIM), jnp.float32),
            pltpu.VMEM((2, CH, EDGE_DIM), jnp.float32),
            pltpu.VMEM((2, CH, EDGE_DIM), jnp.float32),
            pltpu.SemaphoreType.DMA,
            pltpu.SemaphoreType.DMA,
            pltpu.SemaphoreType.DMA,
            pltpu.SemaphoreType.DMA,
        ],
    )(u, v, w, src3, dst3)


NPT = N // SC_NS
NEG = -3e38


def _vg(v, idx):
    return v.at[idx].get(mode="promise_in_bounds")


def _deg_body(dst_hbm, deg_hbm, didx, ones_ch, zb, shared_deg, sem):
    cid = lax.axis_index("c")
    sid = lax.axis_index("s")
    wid = sid * SC_NC + cid
    iota16 = lax.iota(jnp.int32, 16)
    zero16 = jnp.zeros((16,), jnp.float32)
    pltpu.sync_copy(dst_hbm.at[wid], didx)
    for r in range(CH):
        ones_ch[r, :] = jnp.where(iota16 == 0, 1.0, 0.0).astype(jnp.float32)
    for r in range(CH):
        zb[r, :] = zero16
    for q in range(7):
        pltpu.sync_copy(zb, shared_deg.at[pl.ds(sid * NPT + q * CH, CH)])
    pltpu.sync_copy(zb.at[pl.ds(0, NPT - 7 * CH)],
                    shared_deg.at[pl.ds(sid * NPT + 7 * CH, NPT - 7 * CH)])
    plsc.subcore_barrier()

    @pl.loop(0, NCH)
    def _chunk(c):
        pltpu.async_copy(ones_ch, shared_deg.at[didx.at[c]], sem, add=True)

    @pl.loop(0, NCH)
    def _drain(c):
        pltpu.make_async_copy(ones_ch, shared_deg.at[didx.at[c]],
                              sem).wait()

    plsc.subcore_barrier()
    pltpu.sync_copy(shared_deg.at[pl.ds(sid * NPT, NPT)],
                    deg_hbm.at[cid, pl.ds(sid * NPT, NPT)])


def _deg_sc(dst3):
    return pl.kernel(
        _deg_body,
        out_type=jax.ShapeDtypeStruct((SC_NC, N, EDGE_DIM), jnp.float32),
        mesh=_SC_MESH,
        compiler_params=pltpu.CompilerParams(use_tc_tiling_on_sc=False, needs_layout_passes=False),
        scratch_types=[
            pltpu.VMEM((NCH, CH), jnp.int32),
            pltpu.VMEM((CH, EDGE_DIM), jnp.float32),
            pltpu.VMEM((CH, EDGE_DIM), jnp.float32),
            pltpu.VMEM_SHARED((N, EDGE_DIM), jnp.float32),
            pltpu.SemaphoreType.DMA,
        ],
    )(dst3)


def _att_pass1_body(xl_hbm, xr_hbm, ee_hbm, e_hbm, src_hbm, dst_hbm, att_hbm,
                    logit_hbm, mxout_hbm, sume_hbm,
                    sidx, didx, xl_b, xr_b, ee_b, e_b, att_v,
                    logit_b, mx_loc, zb, shared_sum,
                    sem_g0, sem_g1, sem_s0, sem_s1, sem_o):
    cid = lax.axis_index("c")
    sid = lax.axis_index("s")
    wid = sid * SC_NC + cid
    base = wid * EW
    iota16 = lax.iota(jnp.int32, 16)
    zero16 = jnp.zeros((16,), jnp.float32)
    sems = (sem_g0, sem_g1)
    sem_sb = (sem_s0, sem_s1)

    pltpu.sync_copy(src_hbm.at[wid], sidx)
    pltpu.sync_copy(dst_hbm.at[wid], didx)
    pltpu.sync_copy(att_hbm, att_v)

    def issue_loads(b, c):
        pltpu.async_copy(xl_hbm.at[sidx.at[c]], xl_b.at[b], sems[b])
        pltpu.async_copy(xr_hbm.at[didx.at[c]], xr_b.at[b], sems[b])
        pltpu.async_copy(ee_hbm.at[pl.ds(base + c * CH, CH)],
                         ee_b.at[b], sems[b])
        pltpu.async_copy(e_hbm.at[pl.ds(base + c * CH, CH)],
                         e_b.at[b], sems[b])

    def wait_loads(b, c):
        pltpu.make_async_copy(xl_hbm.at[sidx.at[c]], xl_b.at[b],
                              sems[b]).wait()
        pltpu.make_async_copy(xr_hbm.at[didx.at[c]], xr_b.at[b],
                              sems[b]).wait()
        pltpu.make_async_copy(ee_hbm.at[pl.ds(base + c * CH, CH)],
                              ee_b.at[b], sems[b]).wait()
        pltpu.make_async_copy(e_hbm.at[pl.ds(base + c * CH, CH)],
                              e_b.at[b], sems[b]).wait()

    def wait_escatter(b, c):
        pltpu.make_async_copy(e_b.at[b], shared_sum.at[didx.at[c]],
                              sem_sb[b]).wait()

    def compute(b, c):
        pltpu.async_copy(e_b.at[b], shared_sum.at[didx.at[c]],
                         sem_sb[b], add=True)
        for g in range(CH // 16):
            rows_g = iota16 + g * 16
            dst16 = didx[c, pl.ds(g * 16, 16)]
            logits = []
            for h in range(HEADS):
                accs = [zero16, zero16, zero16, zero16]
                att_row = att_v[h, :]
                for l in range(HID):
                    colv = jnp.full((16,), h * HID + l, jnp.int32)
                    a = plsc.load_gather(xl_b.at[b], [rows_g, colv])
                    bb = plsc.load_gather(xr_b.at[b], [rows_g, colv])
                    ce = plsc.load_gather(ee_b.at[b], [rows_g, colv])
                    z = a + bb + ce
                    m = jnp.maximum(z, 0.2 * z)
                    accs[l % 4] = accs[l % 4] + m * att_row[l]
                acc = (accs[0] + accs[1]) + (accs[2] + accs[3])
                logits.append(acc)
                logit_b[b, h, pl.ds(g * 16, 16)] = acc
            skey, perm = plsc.sort_key_val(dst16, iota16)
            eqs = []
            for k in (1, 2, 4, 8):
                imk = jnp.maximum(iota16 - k, 0)
                nb = _vg(skey, imk)
                eqs.append(((skey == nb) & (iota16 >= k), imk))
            nxt = _vg(skey, jnp.minimum(iota16 + 1, 15))
            last = (iota16 == 15) | (skey != nxt)
            s4 = skey * HEADS
            for h in range(HEADS):
                v = _vg(logits[h], perm)
                for eq, imk in eqs:
                    vk = _vg(v, imk)
                    v = jnp.where(eq, jnp.maximum(v, vk), v)
                old = plsc.load_gather(mx_loc, [s4 + h])
                plsc.store_scatter(mx_loc, [s4 + h], jnp.maximum(old, v),
                                   mask=last)
        pltpu.async_copy(logit_b.at[b], logit_hbm.at[wid * NCH + c], sem_o)

    def wait_logit(b, c):
        pltpu.make_async_copy(logit_b.at[b], logit_hbm.at[wid * NCH + c],
                              sem_o).wait()

    for r in range(CH):
        zb[r, :] = zero16
    for q in range(7):
        pltpu.sync_copy(zb, shared_sum.at[pl.ds(sid * NPT + q * CH, CH)])
    pltpu.sync_copy(zb.at[pl.ds(0, NPT - 7 * CH)],
                    shared_sum.at[pl.ds(sid * NPT + 7 * CH, NPT - 7 * CH)])

    @pl.loop(0, N * HEADS // 16)
    def _init_mx(j):
        mx_loc[pl.ds(j * 16, 16)] = jnp.full((16,), NEG, jnp.float32)

    plsc.subcore_barrier()
    issue_loads(0, 0)

    @pl.loop(0, NCH, step=2)
    def _chunk(c):
        @pl.when(c > 0)
        def _():
            wait_logit(0, c - 2)
            wait_logit(1, c - 1)
            wait_escatter(1, c - 1)
        issue_loads(1, c + 1)
        wait_loads(0, c)
        compute(0, c)

        @pl.when(c + 2 < NCH)
        def _():
            wait_escatter(0, c)
            issue_loads(0, c + 2)
        wait_loads(1, c + 1)
        compute(1, c + 1)

    wait_logit(0, NCH - 2)
    wait_logit(1, NCH - 1)
    wait_escatter(0, NCH - 2)
    wait_escatter(1, NCH - 1)
    pltpu.sync_copy(mx_loc, mxout_hbm.at[wid])

    plsc.subcore_barrier()
    pltpu.sync_copy(shared_sum.at[pl.ds(sid * NPT, NPT)],
                    sume_hbm.at[cid, pl.ds(sid * NPT, NPT)])


def _att_pass1_sc(xl, xr, ee, e, src3, dst3, att):
    return pl.kernel(
        _att_pass1_body,
        out_type=(
            jax.ShapeDtypeStruct((SC_NW * NCH, HEADS, CH), jnp.float32),
            jax.ShapeDtypeStruct((SC_NW, N * HEADS), jnp.float32),
            jax.ShapeDtypeStruct((SC_NC, N, EDGE_DIM), jnp.float32),
        ),
        mesh=_SC_MESH,
        compiler_params=pltpu.CompilerParams(use_tc_tiling_on_sc=False, needs_layout_passes=False),
        scratch_types=[
            pltpu.VMEM((NCH, CH), jnp.int32),
            pltpu.VMEM((NCH, CH), jnp.int32),
            pltpu.VMEM((2, CH, NODE_DIM), jnp.float32),
            pltpu.VMEM((2, CH, NODE_DIM), jnp.float32),
            pltpu.VMEM((2, CH, NODE_DIM), jnp.float32),
            pltpu.VMEM((2, CH, EDGE_DIM), jnp.float32),
            pltpu.VMEM((HEADS, HID), jnp.float32),
            pltpu.VMEM((2, HEADS, CH), jnp.float32),
            pltpu.VMEM((N * HEADS,), jnp.float32),
            pltpu.VMEM((CH, EDGE_DIM), jnp.float32),
            pltpu.VMEM_SHARED((N, EDGE_DIM), jnp.float32),
            pltpu.SemaphoreType.DMA,
            pltpu.SemaphoreType.DMA,
            pltpu.SemaphoreType.DMA,
            pltpu.SemaphoreType.DMA,
            pltpu.SemaphoreType.DMA,
        ],
    )(xl, xr, ee, e, src3, dst3, att)


PROD_W = 80


def _att_pass2_body(logit_hbm, mx_hbm, xl_hbm, src_hbm, dst_hbm,
                    out_hbm,
                    sidx, didx, mx_b, xl_b, logit_b, prod_b, zb,
                    shared, sem_g0, sem_g1, sem_s):
    cid = lax.axis_index("c")
    sid = lax.axis_index("s")
    wid = sid * SC_NC + cid
    iota16 = lax.iota(jnp.int32, 16)
    zero16 = jnp.zeros((16,), jnp.float32)
    sems = (sem_g0, sem_g1)

    pltpu.sync_copy(src_hbm.at[wid], sidx)
    pltpu.sync_copy(dst_hbm.at[wid], didx)
    for b in range(2):
        for r in range(CH):
            prod_b[b, r, pl.ds(NODE_DIM, 16)] = zero16
    for r in range(CH):
        for q in range(PROD_W // 16):
            zb[r, pl.ds(q * 16, 16)] = zero16
    for q in range(7):
        pltpu.sync_copy(zb, shared.at[pl.ds(sid * NPT + q * CH, CH)])
    pltpu.sync_copy(zb.at[pl.ds(0, NPT - 7 * CH)],
                    shared.at[pl.ds(sid * NPT + 7 * CH, NPT - 7 * CH)])
    plsc.subcore_barrier()

    def issue_loads(b, c):
        pltpu.async_copy(xl_hbm.at[sidx.at[c]], xl_b.at[b], sems[b])
        pltpu.async_copy(mx_hbm.at[didx.at[c]], mx_b.at[b], sems[b])
        pltpu.async_copy(logit_hbm.at[wid * NCH + c], logit_b.at[b], sems[b])

    def wait_loads(b, c):
        pltpu.make_async_copy(xl_hbm.at[sidx.at[c]], xl_b.at[b],
                              sems[b]).wait()
        pltpu.make_async_copy(mx_hbm.at[didx.at[c]], mx_b.at[b],
                              sems[b]).wait()
        pltpu.make_async_copy(logit_hbm.at[wid * NCH + c], logit_b.at[b],
                              sems[b]).wait()

    def wait_scatter(b, c):
        pltpu.make_async_copy(prod_b.at[b], shared.at[didx.at[c]],
                              sem_s).wait()

    def compute(b, c):
        for g in range(CH // 16):
            rows_g = iota16 + g * 16
            for h in range(HEADS):
                lg = logit_b[b, h, pl.ds(g * 16, 16)]
                mxg = plsc.load_gather(mx_b.at[b],
                                       [rows_g, jnp.full((16,), h, jnp.int32)])
                num = jnp.exp(lg - mxg)
                plsc.store_scatter(prod_b.at[b],
                                   [rows_g, jnp.full((16,), NODE_DIM + h,
                                                     jnp.int32)], num)
        for r in range(CH):
            nums_r = prod_b[b, r, pl.ds(NODE_DIM, 16)]
            for h in range(HEADS):
                prod_b[b, r, pl.ds(h * HID, 16)] = (
                    xl_b[b, r, pl.ds(h * HID, 16)] * nums_r[h])
        pltpu.async_copy(prod_b.at[b], shared.at[didx.at[c]], sem_s,
                         add=True)

    issue_loads(0, 0)

    @pl.loop(0, NCH, step=2)
    def _chunk(c):
        issue_loads(1, c + 1)
        wait_loads(0, c)

        @pl.when(c > 0)
        def _():
            wait_scatter(0, c - 2)
        compute(0, c)

        @pl.when(c + 2 < NCH)
        def _():
            issue_loads(0, c + 2)
        wait_loads(1, c + 1)

        @pl.when(c > 0)
        def _():
            wait_scatter(1, c - 1)
        compute(1, c + 1)

    wait_scatter(0, NCH - 2)
    wait_scatter(1, NCH - 1)
    plsc.subcore_barrier()
    pltpu.sync_copy(shared.at[pl.ds(sid * NPT, NPT)],
                    out_hbm.at[cid, pl.ds(sid * NPT, NPT)])


def _att_pass2_sc(logit, mx_pad, xl, src3, dst3):
    return pl.kernel(
        _att_pass2_body,
        out_type=jax.ShapeDtypeStruct((SC_NC, N, PROD_W), jnp.float32),
        mesh=_SC_MESH,
        compiler_params=pltpu.CompilerParams(use_tc_tiling_on_sc=False, needs_layout_passes=False),
        scratch_types=[
            pltpu.VMEM((NCH, CH), jnp.int32),
            pltpu.VMEM((NCH, CH), jnp.int32),
            pltpu.VMEM((2, CH, EDGE_DIM), jnp.float32),
            pltpu.VMEM((2, CH, NODE_DIM), jnp.float32),
            pltpu.VMEM((2, HEADS, CH), jnp.float32),
            pltpu.VMEM((2, CH, PROD_W), jnp.float32),
            pltpu.VMEM((CH, PROD_W), jnp.float32),
            pltpu.VMEM_SHARED((N, PROD_W), jnp.float32),
            pltpu.SemaphoreType.DMA,
            pltpu.SemaphoreType.DMA,
            pltpu.SemaphoreType.DMA,
        ],
    )(logit, mx_pad, xl, src3, dst3)


EBLK = 8000


def _lr_body(h_ref, wl_ref, bl_ref, wr_ref, br_ref, xl_ref, xr_ref):
    hv = h_ref[...]
    xl_ref[...] = (jnp.dot(hv, wl_ref[...],
                           preferred_element_type=jnp.float32) + bl_ref[...])
    xr_ref[...] = (jnp.dot(hv, wr_ref[...],
                           preferred_element_type=jnp.float32) + br_ref[...])


def _lr_proj(h, wl, bl, wr, br):
    return pl.pallas_call(
        _lr_body,
        out_shape=[jax.ShapeDtypeStruct((N, NODE_DIM), jnp.float32),
                   jax.ShapeDtypeStruct((N, NODE_DIM), jnp.float32)],
    )(h, wl, bl.reshape(1, NODE_DIM), wr, br.reshape(1, NODE_DIM))


def _mm_bias_body(x_ref, w_ref, b_ref, o_ref):
    o_ref[...] = (jnp.dot(x_ref[...], w_ref[...],
                          preferred_element_type=jnp.float32) + b_ref[...])


def _mm_bias(x, wt, b, blk=None):
    m, k = x.shape
    nn = wt.shape[1]
    if blk is None:
        return pl.pallas_call(
            _mm_bias_body,
            out_shape=jax.ShapeDtypeStruct((m, nn), jnp.float32),
        )(x, wt, b.reshape(1, nn))
    return pl.pallas_call(
        _mm_bias_body,
        grid=(m // blk,),
        in_specs=[pl.BlockSpec((blk, k), lambda i: (i, 0)),
                  pl.BlockSpec((k, nn), lambda i: (0, 0)),
                  pl.BlockSpec((1, nn), lambda i: (0, 0))],
        out_specs=pl.BlockSpec((blk, nn), lambda i: (i, 0)),
        out_shape=jax.ShapeDtypeStruct((m, nn), jnp.float32),
    )(x, wt, b.reshape(1, nn))


def _edge_maps_body(e_ref, We_ref, w3_ref, eb_ref, ee_ref, w_ref):
    ev = e_ref[...]
    ee_ref[...] = jnp.dot(ev, We_ref[...], preferred_element_type=jnp.float32)
    w_ref[...] = (jnp.dot(ev, w3_ref[...], preferred_element_type=jnp.float32)
                  + eb_ref[...])


def _edge_maps(e, We, w3, eb):
    return pl.pallas_call(
        _edge_maps_body,
        grid=(E // EBLK,),
        in_specs=[pl.BlockSpec((EBLK, EDGE_DIM), lambda i: (i, 0)),
                  pl.BlockSpec((EDGE_DIM, NODE_DIM), lambda i: (0, 0)),
                  pl.BlockSpec((EDGE_DIM, EDGE_DIM), lambda i: (0, 0)),
                  pl.BlockSpec((1, EDGE_DIM), lambda i: (0, 0))],
        out_specs=[pl.BlockSpec((EBLK, NODE_DIM), lambda i: (i, 0)),
                   pl.BlockSpec((EBLK, EDGE_DIM), lambda i: (i, 0))],
        out_shape=[jax.ShapeDtypeStruct((E, NODE_DIM), jnp.float32),
                   jax.ShapeDtypeStruct((E, EDGE_DIM), jnp.float32)],
    )(e, We, w3, eb.reshape(1, EDGE_DIM))


def _node_mid_body(sume_ref, deg_ref, xl_ref, xr_ref, We_ref, attf_ref,
                   ls_ref):
    deg = jnp.maximum(deg_ref[...], 1.0)
    loop_attr = sume_ref[...] / deg
    ee_self = jnp.dot(loop_attr, We_ref[...],
                      preferred_element_type=jnp.float32)
    z = xl_ref[...] + xr_ref[...] + ee_self
    m = jnp.maximum(z, 0.2 * z) * attf_ref[...]
    sel = (lax.broadcasted_iota(jnp.int32, (NODE_DIM, HEADS), 0) // HID
           == lax.broadcasted_iota(jnp.int32, (NODE_DIM, HEADS), 1)
           ).astype(jnp.float32)
    ls_ref[...] = jnp.dot(m, sel, preferred_element_type=jnp.float32)


def _node_mid(sum_e, deg, x_l, x_r, We, att):
    return pl.pallas_call(
        _node_mid_body,
        out_shape=jax.ShapeDtypeStruct((N, HEADS), jnp.float32),
    )(sum_e, deg.reshape(N, 1), x_l, x_r, We,
      att.reshape(1, NODE_DIM))


def _node_post_body(aggp_ref, denp_ref, xl_ref, nums_ref, bias_ref,
                    ng_ref, nb_ref, eW1_ref, eW2_ref,
                    hn_ref, u_ref, v_ref):
    rep = (lax.broadcasted_iota(jnp.int32, (HEADS, NODE_DIM), 0)
           == lax.broadcasted_iota(jnp.int32, (HEADS, NODE_DIM), 1) // HID
           ).astype(jnp.float32)
    num_rep = jnp.dot(nums_ref[...], rep,
                      preferred_element_type=jnp.float32)
    den_rep = jnp.dot(denp_ref[...] + nums_ref[...], rep,
                      preferred_element_type=jnp.float32)
    agg = (aggp_ref[...] + xl_ref[...] * num_rep) / den_rep
    hn = _ln(_silu(agg + bias_ref[...]), ng_ref[...], nb_ref[...])
    hn_ref[...] = hn
    u_ref[...] = jnp.dot(hn, eW1_ref[...], preferred_element_type=jnp.float32)
    v_ref[...] = jnp.dot(hn, eW2_ref[...], preferred_element_type=jnp.float32)


def _node_post(aggp, denp, x_l, num_s, bias, ng, nb, eW1, eW2):
    return pl.pallas_call(
        _node_post_body,
        out_shape=[jax.ShapeDtypeStruct((N, NODE_DIM), jnp.float32),
                   jax.ShapeDtypeStruct((N, EDGE_DIM), jnp.float32),
                   jax.ShapeDtypeStruct((N, EDGE_DIM), jnp.float32)],
    )(aggp, denp, x_l, num_s, bias.reshape(1, NODE_DIM),
      ng.reshape(1, NODE_DIM), nb.reshape(1, NODE_DIM), eW1, eW2)


def _edge_ln_body(es_ref, g_ref, b_ref, o_ref):
    o_ref[...] = _ln(_silu(es_ref[...]), g_ref[...], b_ref[...])


def _edge_ln(esum, g, b):
    return pl.pallas_call(
        _edge_ln_body,
        grid=(E // EBLK,),
        in_specs=[pl.BlockSpec((EBLK, EDGE_DIM), lambda i: (i, 0)),
                  pl.BlockSpec((1, EDGE_DIM), lambda i: (0, 0)),
                  pl.BlockSpec((1, EDGE_DIM), lambda i: (0, 0))],
        out_specs=pl.BlockSpec((EBLK, EDGE_DIM), lambda i: (i, 0)),
        out_shape=jax.ShapeDtypeStruct((E, EDGE_DIM), jnp.float32),
    )(esum, g.reshape(1, EDGE_DIM), b.reshape(1, EDGE_DIM))


def _pool_body(hjk_ref, batch_ref, jkW_ref, jkb_ref, jkg_ref, jkbt_ref,
               gW1_ref, gb1_ref, gW2_ref, gb2_ref, hW_ref, hb_ref, out_ref):
    hjk = hjk_ref[...]
    h = jnp.dot(hjk, jkW_ref[...], preferred_element_type=jnp.float32)
    h = h + jkb_ref[...]
    h = _ln(_silu(h), jkg_ref[...], jkbt_ref[...])
    g1 = _silu(jnp.dot(h, gW1_ref[...], preferred_element_type=jnp.float32)
               + gb1_ref[...])
    gate = (jnp.dot(g1, gW2_ref[...], preferred_element_type=jnp.float32)
            + gb2_ref[...])[:, 0]
    batch = batch_ref[0, :]
    seg = lax.broadcasted_iota(jnp.int32, (G, N), 0)
    mask = (batch[None, :] == seg)
    neg = jnp.float32(-3e38)
    gm = jnp.max(jnp.where(mask, gate[None, :], neg), axis=1)
    gm_n = jnp.sum(jnp.where(mask, gm[:, None], 0.0), axis=0)
    gnum = jnp.exp(gate - gm_n)
    gden = jnp.sum(jnp.where(mask, gnum[None, :], 0.0), axis=1)
    gden_n = jnp.sum(jnp.where(mask, gden[:, None], 0.0), axis=0)
    a = gnum / gden_n
    wmask = jnp.where(mask, a[None, :], 0.0)
    hg = jnp.dot(wmask, h, preferred_element_type=jnp.float32)
    out_ref[...] = (jnp.dot(hg, hW_ref[...], preferred_element_type=jnp.float32)
                    + hb_ref[...])


def _pool_head(hjk, batch, params):
    return pl.pallas_call(
        _pool_body,
        out_shape=jax.ShapeDtypeStruct((G, 1), jnp.float32),
    )(hjk, batch.reshape(1, N).astype(jnp.int32),
      params['jk_W'], params['jk_b'].reshape(1, NODE_DIM),
      params['jk_g'].reshape(1, NODE_DIM), params['jk_bt'].reshape(1, NODE_DIM),
      params['g_W1'], params['g_b1'].reshape(1, NODE_DIM // 2),
      params['g_W2'], params['g_b2'].reshape(1, 1),
      params['head_W'], params['head_b'].reshape(1, 1))


def kernel(x, edge_index, edge_attr, batch, params):
    n = N
    src, dst = edge_index[0], edge_index[1]
    src3 = src.astype(jnp.int32).reshape(SC_NW, NCH, CH)
    dst3 = dst.astype(jnp.int32).reshape(SC_NW, NCH, CH)
    h = _mm_bias(x, params['atom_W'], params['atom_b'])
    e = _mm_bias(edge_attr, params['bond_W'], params['bond_b'], blk=EBLK)
    deg = _deg_sc(dst3).sum(axis=0)[:, 0]
    outs = []
    for lp in params['layers']:
        att2 = lp['att'].reshape(HEADS, HID)
        x_l, x_r = _lr_proj(h, lp['Wl'], lp['bl'], lp['Wr'], lp['br'])
        ee, w = _edge_maps(e, lp['We'], lp['eW'][2 * NODE_DIM:], lp['eb'])
        logit_e, mx_parts, sume_parts = _att_pass1_sc(
            x_l, x_r, ee, e, src3, dst3, att2)
        sum_e = sume_parts.sum(axis=0)
        logit_s = _node_mid(sum_e, deg, x_l, x_r, lp['We'], lp['att'])
        mx = jnp.maximum(mx_parts.reshape(SC_NW, n, HEADS).max(axis=0),
                         logit_s)
        num_s = jnp.exp(logit_s - mx)
        mx_pad = jnp.pad(mx, ((0, 0), (0, EDGE_DIM - HEADS)))
        part = _att_pass2_sc(logit_e, mx_pad, x_l, src3, dst3)
        aggp = part[:, :, :NODE_DIM].sum(axis=0)
        denp = part[:, :, NODE_DIM:NODE_DIM + HEADS].sum(axis=0)
        hn, u, v = _node_post(aggp, denp, x_l, num_s, lp['bias'],
                              lp['ng'], lp['nb'], lp['eW'][:NODE_DIM],
                              lp['eW'][NODE_DIM:2 * NODE_DIM])
        esum = _edge_sum_sc(u, v, w, src3, dst3)
        e = _edge_ln(esum, lp['eg'], lp['ebt'])
        h = hn
        outs.append(h)
    hjk = jnp.concatenate(outs, axis=-1)
    return _pool_head(hjk, batch, params)


# speedup vs baseline: 1.4916x; 1.4916x over previous
import functools
import jax
import jax.numpy as jnp
from jax import lax
from jax.experimental import pallas as pl
from jax.experimental.pallas import tpu as pltpu
from jax.experimental.pallas import tpu_sc as plsc

N = 10000
E = 640000
D_IN = 128
EDGE_DIM = 16
HEADS = 4
HID = 16
NODE_DIM = HEADS * HID
L = 4
G = 64


def _silu(x):
    return x * jax.nn.sigmoid(x)


def _ln(x, g, b):
    mu = jnp.mean(x, axis=-1, keepdims=True)
    var = jnp.var(x, axis=-1, keepdims=True)
    return (x - mu) / jnp.sqrt(var + 1e-5) * g + b


SC_NC = 2
SC_NS = 16
SC_NW = SC_NC * SC_NS
EW = E // SC_NW
CH = 80
NCH = EW // CH

_SC_MESH = plsc.VectorSubcoreMesh(
    core_axis_name="c", subcore_axis_name="s",
    num_cores=SC_NC, num_subcores=SC_NS)


def _wid():
    return lax.axis_index("s") * SC_NC + lax.axis_index("c")


def _edge_sum_body(u_hbm, v_hbm, w_hbm, src_hbm, dst_hbm, out_hbm,
                   sidx, didx, u_b, v_b, w_b, ov_b,
                   sem_g0, sem_g1, sem_o0, sem_o1):
    wid = _wid()
    base = wid * EW
    sems = (sem_g0, sem_g1)
    sem_ob = (sem_o0, sem_o1)
    pltpu.sync_copy(src_hbm.at[wid], sidx)
    pltpu.sync_copy(dst_hbm.at[wid], didx)

    def issue_loads(b, c):
        pltpu.async_copy(u_hbm.at[sidx.at[c]], u_b.at[b], sems[b])
        pltpu.async_copy(v_hbm.at[didx.at[c]], v_b.at[b], sems[b])
        pltpu.async_copy(w_hbm.at[pl.ds(base + c * CH, CH)],
                         w_b.at[b], sems[b])

    def wait_loads(b, c):
        pltpu.make_async_copy(u_hbm.at[sidx.at[c]], u_b.at[b],
                              sems[b]).wait()
        pltpu.make_async_copy(v_hbm.at[didx.at[c]], v_b.at[b],
                              sems[b]).wait()
        pltpu.make_async_copy(w_hbm.at[pl.ds(base + c * CH, CH)],
                              w_b.at[b], sems[b]).wait()

    def wait_out(b, c):
        pltpu.make_async_copy(ov_b.at[b],
                              out_hbm.at[pl.ds(base + c * CH, CH)],
                              sem_ob[b]).wait()

    def compute(b, c):
        for r in range(CH):
            ov_b[b, r, :] = u_b[b, r, :] + v_b[b, r, :] + w_b[b, r, :]
        pltpu.async_copy(ov_b.at[b], out_hbm.at[pl.ds(base + c * CH, CH)],
                         sem_ob[b])

    issue_loads(0, 0)

    @pl.loop(0, NCH, step=2)
    def _chunk(c):
        @pl.when(c > 0)
        def _():
            wait_out(1, c - 1)
        issue_loads(1, c + 1)
        wait_loads(0, c)

        @pl.when(c > 0)
        def _():
            wait_out(0, c - 2)
        compute(0, c)

        @pl.when(c + 2 < NCH)
        def _():
            issue_loads(0, c + 2)
        wait_loads(1, c + 1)
        compute(1, c + 1)

    wait_out(0, NCH - 2)
    wait_out(1, NCH - 1)


def _edge_sum_sc(u, v, w, src3, dst3):
    return pl.kernel(
        _edge_sum_body,
        out_type=jax.ShapeDtypeStruct((E, EDGE_DIM), jnp.float32),
        mesh=_SC_MESH,
        compiler_params=pltpu.CompilerParams(use_tc_tiling_on_sc=False, needs_layout_passes=False),
        scratch_types=[
            pltpu.VMEM((NCH, CH), jnp.int32),
            pltpu.VMEM((NCH, CH), jnp.int32),
            pltpu.VMEM((2, CH, EDGE_DIM), jnp.float32),
            pltpu.VMEM((2, CH, EDGE_DIM), jnp.float32),
            pltpu.VMEM((2, CH, EDGE_DIM), jnp.float32),
            pltpu.VMEM((2, CH, EDGE_DIM), jnp.float32),
            pltpu.SemaphoreType.DMA,
            pltpu.SemaphoreType.DMA,
            pltpu.SemaphoreType.DMA,
            pltpu.SemaphoreType.DMA,
        ],
    )(u, v, w, src3, dst3)


NPT = N // SC_NS
NEG = -3e38


def _vg(v, idx):
    return v.at[idx].get(mode="promise_in_bounds")


def _deg_body(dst_hbm, deg_hbm, didx, ones_ch, zb, shared_deg, sem):
    cid = lax.axis_index("c")
    sid = lax.axis_index("s")
    wid = sid * SC_NC + cid
    iota16 = lax.iota(jnp.int32, 16)
    zero16 = jnp.zeros((16,), jnp.float32)
    pltpu.sync_copy(dst_hbm.at[wid], didx)
    for r in range(CH):
        ones_ch[r, :] = jnp.where(iota16 == 0, 1.0, 0.0).astype(jnp.float32)
    for r in range(CH):
        zb[r, :] = zero16
    for q in range(7):
        pltpu.sync_copy(zb, shared_deg.at[pl.ds(sid * NPT + q * CH, CH)])
    pltpu.sync_copy(zb.at[pl.ds(0, NPT - 7 * CH)],
                    shared_deg.at[pl.ds(sid * NPT + 7 * CH, NPT - 7 * CH)])
    plsc.subcore_barrier()

    @pl.loop(0, NCH)
    def _chunk(c):
        pltpu.async_copy(ones_ch, shared_deg.at[didx.at[c]], sem, add=True)

    @pl.loop(0, NCH)
    def _drain(c):
        pltpu.make_async_copy(ones_ch, shared_deg.at[didx.at[c]],
                              sem).wait()

    plsc.subcore_barrier()
    pltpu.sync_copy(shared_deg.at[pl.ds(sid * NPT, NPT)],
                    deg_hbm.at[cid, pl.ds(sid * NPT, NPT)])


def _deg_sc(dst3):
    return pl.kernel(
        _deg_body,
        out_type=jax.ShapeDtypeStruct((SC_NC, N, EDGE_DIM), jnp.float32),
        mesh=_SC_MESH,
        compiler_params=pltpu.CompilerParams(use_tc_tiling_on_sc=False, needs_layout_passes=False),
        scratch_types=[
            pltpu.VMEM((NCH, CH), jnp.int32),
            pltpu.VMEM((CH, EDGE_DIM), jnp.float32),
            pltpu.VMEM((CH, EDGE_DIM), jnp.float32),
            pltpu.VMEM_SHARED((N, EDGE_DIM), jnp.float32),
            pltpu.SemaphoreType.DMA,
        ],
    )(dst3)


def _att_pass1_body(xl_hbm, xr_hbm, ee_hbm, e_hbm, src_hbm, dst_hbm, att_hbm,
                    logit_hbm, mxout_hbm, sume_hbm,
                    sidx, didx, xl_b, xr_b, ee_b, e_b, att_v,
                    logit_b, mx_loc, zb, shared_sum,
                    sem_g0, sem_g1, sem_s0, sem_s1, sem_o):
    cid = lax.axis_index("c")
    sid = lax.axis_index("s")
    wid = sid * SC_NC + cid
    base = wid * EW
    iota16 = lax.iota(jnp.int32, 16)
    zero16 = jnp.zeros((16,), jnp.float32)
    sems = (sem_g0, sem_g1)
    sem_sb = (sem_s0, sem_s1)

    pltpu.sync_copy(src_hbm.at[wid], sidx)
    pltpu.sync_copy(dst_hbm.at[wid], didx)
    pltpu.sync_copy(att_hbm, att_v)

    def issue_loads(b, c):
        pltpu.async_copy(xl_hbm.at[sidx.at[c]], xl_b.at[b], sems[b])
        pltpu.async_copy(xr_hbm.at[didx.at[c]], xr_b.at[b], sems[b])
        pltpu.async_copy(ee_hbm.at[pl.ds(base + c * CH, CH)],
                         ee_b.at[b], sems[b])
        pltpu.async_copy(e_hbm.at[pl.ds(base + c * CH, CH)],
                         e_b.at[b], sems[b])

    def wait_loads(b, c):
        pltpu.make_async_copy(xl_hbm.at[sidx.at[c]], xl_b.at[b],
                              sems[b]).wait()
        pltpu.make_async_copy(xr_hbm.at[didx.at[c]], xr_b.at[b],
                              sems[b]).wait()
        pltpu.make_async_copy(ee_hbm.at[pl.ds(base + c * CH, CH)],
                              ee_b.at[b], sems[b]).wait()
        pltpu.make_async_copy(e_hbm.at[pl.ds(base + c * CH, CH)],
                              e_b.at[b], sems[b]).wait()

    def wait_escatter(b, c):
        pltpu.make_async_copy(e_b.at[b], shared_sum.at[didx.at[c]],
                              sem_sb[b]).wait()

    def compute(b, c):
        pltpu.async_copy(e_b.at[b], shared_sum.at[didx.at[c]],
                         sem_sb[b], add=True)
        @pl.loop(0, CH)
        def _presum(r):
            for q in range(HEADS):
                sl = pl.ds(q * HID, 16)
                xl_b[b, r, sl] = (xl_b[b, r, sl] + xr_b[b, r, sl]
                                  + ee_b[b, r, sl])
        for g in range(CH // 16):
            rows_g = iota16 + g * 16
            dst16 = didx[c, pl.ds(g * 16, 16)]
            logits = []
            for h in range(HEADS):
                accs = [zero16, zero16, zero16, zero16]
                att_row = att_v[h, :]
                for l in range(HID):
                    colv = jnp.full((16,), h * HID + l, jnp.int32)
                    z = plsc.load_gather(xl_b.at[b], [rows_g, colv])
                    m = jnp.maximum(z, 0.2 * z)
                    accs[l % 4] = accs[l % 4] + m * att_row[l]
                acc = (accs[0] + accs[1]) + (accs[2] + accs[3])
                logits.append(acc)
                logit_b[b, h, pl.ds(g * 16, 16)] = acc
            skey, perm = plsc.sort_key_val(dst16, iota16)
            eqs = []
            for k in (1, 2, 4, 8):
                imk = jnp.maximum(iota16 - k, 0)
                nb = _vg(skey, imk)
                eqs.append(((skey == nb) & (iota16 >= k), imk))
            nxt = _vg(skey, jnp.minimum(iota16 + 1, 15))
            last = (iota16 == 15) | (skey != nxt)
            s4 = skey * HEADS
            for h in range(HEADS):
                v = _vg(logits[h], perm)
                for eq, imk in eqs:
                    vk = _vg(v, imk)
                    v = jnp.where(eq, jnp.maximum(v, vk), v)
                old = plsc.load_gather(mx_loc, [s4 + h])
                plsc.store_scatter(mx_loc, [s4 + h], jnp.maximum(old, v),
                                   mask=last)
        pltpu.async_copy(logit_b.at[b], logit_hbm.at[wid * NCH + c], sem_o)

    def wait_logit(b, c):
        pltpu.make_async_copy(logit_b.at[b], logit_hbm.at[wid * NCH + c],
                              sem_o).wait()

    for r in range(CH):
        zb[r, :] = zero16
    for q in range(7):
        pltpu.sync_copy(zb, shared_sum.at[pl.ds(sid * NPT + q * CH, CH)])
    pltpu.sync_copy(zb.at[pl.ds(0, NPT - 7 * CH)],
                    shared_sum.at[pl.ds(sid * NPT + 7 * CH, NPT - 7 * CH)])

    @pl.loop(0, N * HEADS // 16)
    def _init_mx(j):
        mx_loc[pl.ds(j * 16, 16)] = jnp.full((16,), NEG, jnp.float32)

    plsc.subcore_barrier()
    issue_loads(0, 0)

    @pl.loop(0, NCH, step=2)
    def _chunk(c):
        @pl.when(c > 0)
        def _():
            wait_logit(0, c - 2)
            wait_logit(1, c - 1)
            wait_escatter(1, c - 1)
        issue_loads(1, c + 1)
        wait_loads(0, c)
        compute(0, c)

        @pl.when(c + 2 < NCH)
        def _():
            wait_escatter(0, c)
            issue_loads(0, c + 2)
        wait_loads(1, c + 1)
        compute(1, c + 1)

    wait_logit(0, NCH - 2)
    wait_logit(1, NCH - 1)
    wait_escatter(0, NCH - 2)
    wait_escatter(1, NCH - 1)
    pltpu.sync_copy(mx_loc, mxout_hbm.at[wid])

    plsc.subcore_barrier()
    pltpu.sync_copy(shared_sum.at[pl.ds(sid * NPT, NPT)],
                    sume_hbm.at[cid, pl.ds(sid * NPT, NPT)])


def _att_pass1_sc(xl, xr, ee, e, src3, dst3, att):
    return pl.kernel(
        _att_pass1_body,
        out_type=(
            jax.ShapeDtypeStruct((SC_NW * NCH, HEADS, CH), jnp.float32),
            jax.ShapeDtypeStruct((SC_NW, N * HEADS), jnp.float32),
            jax.ShapeDtypeStruct((SC_NC, N, EDGE_DIM), jnp.float32),
        ),
        mesh=_SC_MESH,
        compiler_params=pltpu.CompilerParams(use_tc_tiling_on_sc=False, needs_layout_passes=False),
        scratch_types=[
            pltpu.VMEM((NCH, CH), jnp.int32),
            pltpu.VMEM((NCH, CH), jnp.int32),
            pltpu.VMEM((2, CH, NODE_DIM), jnp.float32),
            pltpu.VMEM((2, CH, NODE_DIM), jnp.float32),
            pltpu.VMEM((2, CH, NODE_DIM), jnp.float32),
            pltpu.VMEM((2, CH, EDGE_DIM), jnp.float32),
            pltpu.VMEM((HEADS, HID), jnp.float32),
            pltpu.VMEM((2, HEADS, CH), jnp.float32),
            pltpu.VMEM((N * HEADS,), jnp.float32),
            pltpu.VMEM((CH, EDGE_DIM), jnp.float32),
            pltpu.VMEM_SHARED((N, EDGE_DIM), jnp.float32),
            pltpu.SemaphoreType.DMA,
            pltpu.SemaphoreType.DMA,
            pltpu.SemaphoreType.DMA,
            pltpu.SemaphoreType.DMA,
            pltpu.SemaphoreType.DMA,
        ],
    )(xl, xr, ee, e, src3, dst3, att)


PROD_W = 80


def _att_pass2_body(logit_hbm, mx_hbm, xl_hbm, src_hbm, dst_hbm,
                    out_hbm,
                    sidx, didx, mx_b, xl_b, logit_b, prod_b, zb,
                    shared, sem_g0, sem_g1, sem_s):
    cid = lax.axis_index("c")
    sid = lax.axis_index("s")
    wid = sid * SC_NC + cid
    iota16 = lax.iota(jnp.int32, 16)
    zero16 = jnp.zeros((16,), jnp.float32)
    sems = (sem_g0, sem_g1)

    pltpu.sync_copy(src_hbm.at[wid], sidx)
    pltpu.sync_copy(dst_hbm.at[wid], didx)
    for b in range(2):
        for r in range(CH):
            prod_b[b, r, pl.ds(NODE_DIM, 16)] = zero16
    for r in range(CH):
        for q in range(PROD_W // 16):
            zb[r, pl.ds(q * 16, 16)] = zero16
    for q in range(7):
        pltpu.sync_copy(zb, shared.at[pl.ds(sid * NPT + q * CH, CH)])
    pltpu.sync_copy(zb.at[pl.ds(0, NPT - 7 * CH)],
                    shared.at[pl.ds(sid * NPT + 7 * CH, NPT - 7 * CH)])
    plsc.subcore_barrier()

    def issue_loads(b, c):
        pltpu.async_copy(xl_hbm.at[sidx.at[c]], xl_b.at[b], sems[b])
        pltpu.async_copy(mx_hbm.at[didx.at[c]], mx_b.at[b], sems[b])
        pltpu.async_copy(logit_hbm.at[wid * NCH + c], logit_b.at[b], sems[b])

    def wait_loads(b, c):
        pltpu.make_async_copy(xl_hbm.at[sidx.at[c]], xl_b.at[b],
                              sems[b]).wait()
        pltpu.make_async_copy(mx_hbm.at[didx.at[c]], mx_b.at[b],
                              sems[b]).wait()
        pltpu.make_async_copy(logit_hbm.at[wid * NCH + c], logit_b.at[b],
                              sems[b]).wait()

    def wait_scatter(b, c):
        pltpu.make_async_copy(prod_b.at[b], shared.at[didx.at[c]],
                              sem_s).wait()

    def compute(b, c):
        for g in range(CH // 16):
            rows_g = iota16 + g * 16
            for h in range(HEADS):
                lg = logit_b[b, h, pl.ds(g * 16, 16)]
                mxg = plsc.load_gather(mx_b.at[b],
                                       [rows_g, jnp.full((16,), h, jnp.int32)])
                num = jnp.exp(lg - mxg)
                plsc.store_scatter(prod_b.at[b],
                                   [rows_g, jnp.full((16,), NODE_DIM + h,
                                                     jnp.int32)], num)
        for r in range(CH):
            nums_r = prod_b[b, r, pl.ds(NODE_DIM, 16)]
            for h in range(HEADS):
                prod_b[b, r, pl.ds(h * HID, 16)] = (
                    xl_b[b, r, pl.ds(h * HID, 16)] * nums_r[h])
        pltpu.async_copy(prod_b.at[b], shared.at[didx.at[c]], sem_s,
                         add=True)

    issue_loads(0, 0)

    @pl.loop(0, NCH, step=2)
    def _chunk(c):
        issue_loads(1, c + 1)
        wait_loads(0, c)

        @pl.when(c > 0)
        def _():
            wait_scatter(0, c - 2)
        compute(0, c)

        @pl.when(c + 2 < NCH)
        def _():
            issue_loads(0, c + 2)
        wait_loads(1, c + 1)

        @pl.when(c > 0)
        def _():
            wait_scatter(1, c - 1)
        compute(1, c + 1)

    wait_scatter(0, NCH - 2)
    wait_scatter(1, NCH - 1)
    plsc.subcore_barrier()
    pltpu.sync_copy(shared.at[pl.ds(sid * NPT, NPT)],
                    out_hbm.at[cid, pl.ds(sid * NPT, NPT)])


def _att_pass2_sc(logit, mx_pad, xl, src3, dst3):
    return pl.kernel(
        _att_pass2_body,
        out_type=jax.ShapeDtypeStruct((SC_NC, N, PROD_W), jnp.float32),
        mesh=_SC_MESH,
        compiler_params=pltpu.CompilerParams(use_tc_tiling_on_sc=False, needs_layout_passes=False),
        scratch_types=[
            pltpu.VMEM((NCH, CH), jnp.int32),
            pltpu.VMEM((NCH, CH), jnp.int32),
            pltpu.VMEM((2, CH, EDGE_DIM), jnp.float32),
            pltpu.VMEM((2, CH, NODE_DIM), jnp.float32),
            pltpu.VMEM((2, HEADS, CH), jnp.float32),
            pltpu.VMEM((2, CH, PROD_W), jnp.float32),
            pltpu.VMEM((CH, PROD_W), jnp.float32),
            pltpu.VMEM_SHARED((N, PROD_W), jnp.float32),
            pltpu.SemaphoreType.DMA,
            pltpu.SemaphoreType.DMA,
            pltpu.SemaphoreType.DMA,
        ],
    )(logit, mx_pad, xl, src3, dst3)


EBLK = 8000


def _lr_body(h_ref, wl_ref, bl_ref, wr_ref, br_ref, xl_ref, xr_ref):
    hv = h_ref[...]
    xl_ref[...] = (jnp.dot(hv, wl_ref[...],
                           preferred_element_type=jnp.float32) + bl_ref[...])
    xr_ref[...] = (jnp.dot(hv, wr_ref[...],
                           preferred_element_type=jnp.float32) + br_ref[...])


def _lr_proj(h, wl, bl, wr, br):
    return pl.pallas_call(
        _lr_body,
        out_shape=[jax.ShapeDtypeStruct((N, NODE_DIM), jnp.float32),
                   jax.ShapeDtypeStruct((N, NODE_DIM), jnp.float32)],
    )(h, wl, bl.reshape(1, NODE_DIM), wr, br.reshape(1, NODE_DIM))


def _mm_bias_body(x_ref, w_ref, b_ref, o_ref):
    o_ref[...] = (jnp.dot(x_ref[...], w_ref[...],
                          preferred_element_type=jnp.float32) + b_ref[...])


def _mm_bias(x, wt, b, blk=None):
    m, k = x.shape
    nn = wt.shape[1]
    if blk is None:
        return pl.pallas_call(
            _mm_bias_body,
            out_shape=jax.ShapeDtypeStruct((m, nn), jnp.float32),
        )(x, wt, b.reshape(1, nn))
    return pl.pallas_call(
        _mm_bias_body,
        grid=(m // blk,),
        in_specs=[pl.BlockSpec((blk, k), lambda i: (i, 0)),
                  pl.BlockSpec((k, nn), lambda i: (0, 0)),
                  pl.BlockSpec((1, nn), lambda i: (0, 0))],
        out_specs=pl.BlockSpec((blk, nn), lambda i: (i, 0)),
        out_shape=jax.ShapeDtypeStruct((m, nn), jnp.float32),
    )(x, wt, b.reshape(1, nn))


def _edge_maps_body(e_ref, We_ref, w3_ref, eb_ref, ee_ref, w_ref):
    ev = e_ref[...]
    ee_ref[...] = jnp.dot(ev, We_ref[...], preferred_element_type=jnp.float32)
    w_ref[...] = (jnp.dot(ev, w3_ref[...], preferred_element_type=jnp.float32)
                  + eb_ref[...])


def _edge_maps(e, We, w3, eb):
    return pl.pallas_call(
        _edge_maps_body,
        grid=(E // EBLK,),
        in_specs=[pl.BlockSpec((EBLK, EDGE_DIM), lambda i: (i, 0)),
                  pl.BlockSpec((EDGE_DIM, NODE_DIM), lambda i: (0, 0)),
                  pl.BlockSpec((EDGE_DIM, EDGE_DIM), lambda i: (0, 0)),
                  pl.BlockSpec((1, EDGE_DIM), lambda i: (0, 0))],
        out_specs=[pl.BlockSpec((EBLK, NODE_DIM), lambda i: (i, 0)),
                   pl.BlockSpec((EBLK, EDGE_DIM), lambda i: (i, 0))],
        out_shape=[jax.ShapeDtypeStruct((E, NODE_DIM), jnp.float32),
                   jax.ShapeDtypeStruct((E, EDGE_DIM), jnp.float32)],
    )(e, We, w3, eb.reshape(1, EDGE_DIM))


def _node_mid_body(sume_ref, deg_ref, xl_ref, xr_ref, We_ref, attf_ref,
                   ls_ref):
    deg = jnp.maximum(deg_ref[...], 1.0)
    loop_attr = sume_ref[...] / deg
    ee_self = jnp.dot(loop_attr, We_ref[...],
                      preferred_element_type=jnp.float32)
    z = xl_ref[...] + xr_ref[...] + ee_self
    m = jnp.maximum(z, 0.2 * z) * attf_ref[...]
    sel = (lax.broadcasted_iota(jnp.int32, (NODE_DIM, HEADS), 0) // HID
           == lax.broadcasted_iota(jnp.int32, (NODE_DIM, HEADS), 1)
           ).astype(jnp.float32)
    ls_ref[...] = jnp.dot(m, sel, preferred_element_type=jnp.float32)


def _node_mid(sum_e, deg, x_l, x_r, We, att):
    return pl.pallas_call(
        _node_mid_body,
        out_shape=jax.ShapeDtypeStruct((N, HEADS), jnp.float32),
    )(sum_e, deg.reshape(N, 1), x_l, x_r, We,
      att.reshape(1, NODE_DIM))


def _node_post_body(aggp_ref, denp_ref, xl_ref, nums_ref, bias_ref,
                    ng_ref, nb_ref, eW1_ref, eW2_ref,
                    hn_ref, u_ref, v_ref):
    rep = (lax.broadcasted_iota(jnp.int32, (HEADS, NODE_DIM), 0)
           == lax.broadcasted_iota(jnp.int32, (HEADS, NODE_DIM), 1) // HID
           ).astype(jnp.float32)
    num_rep = jnp.dot(nums_ref[...], rep,
                      preferred_element_type=jnp.float32)
    den_rep = jnp.dot(denp_ref[...] + nums_ref[...], rep,
                      preferred_element_type=jnp.float32)
    agg = (aggp_ref[...] + xl_ref[...] * num_rep) / den_rep
    hn = _ln(_silu(agg + bias_ref[...]), ng_ref[...], nb_ref[...])
    hn_ref[...] = hn
    u_ref[...] = jnp.dot(hn, eW1_ref[...], preferred_element_type=jnp.float32)
    v_ref[...] = jnp.dot(hn, eW2_ref[...], preferred_element_type=jnp.float32)


def _node_post(aggp, denp, x_l, num_s, bias, ng, nb, eW1, eW2):
    return pl.pallas_call(
        _node_post_body,
        out_shape=[jax.ShapeDtypeStruct((N, NODE_DIM), jnp.float32),
                   jax.ShapeDtypeStruct((N, EDGE_DIM), jnp.float32),
                   jax.ShapeDtypeStruct((N, EDGE_DIM), jnp.float32)],
    )(aggp, denp, x_l, num_s, bias.reshape(1, NODE_DIM),
      ng.reshape(1, NODE_DIM), nb.reshape(1, NODE_DIM), eW1, eW2)


def _edge_ln_body(es_ref, g_ref, b_ref, o_ref):
    o_ref[...] = _ln(_silu(es_ref[...]), g_ref[...], b_ref[...])


def _edge_ln(esum, g, b):
    return pl.pallas_call(
        _edge_ln_body,
        grid=(E // EBLK,),
        in_specs=[pl.BlockSpec((EBLK, EDGE_DIM), lambda i: (i, 0)),
                  pl.BlockSpec((1, EDGE_DIM), lambda i: (0, 0)),
                  pl.BlockSpec((1, EDGE_DIM), lambda i: (0, 0))],
        out_specs=pl.BlockSpec((EBLK, EDGE_DIM), lambda i: (i, 0)),
        out_shape=jax.ShapeDtypeStruct((E, EDGE_DIM), jnp.float32),
    )(esum, g.reshape(1, EDGE_DIM), b.reshape(1, EDGE_DIM))


def _pool_body(hjk_ref, batch_ref, jkW_ref, jkb_ref, jkg_ref, jkbt_ref,
               gW1_ref, gb1_ref, gW2_ref, gb2_ref, hW_ref, hb_ref, out_ref):
    hjk = hjk_ref[...]
    h = jnp.dot(hjk, jkW_ref[...], preferred_element_type=jnp.float32)
    h = h + jkb_ref[...]
    h = _ln(_silu(h), jkg_ref[...], jkbt_ref[...])
    g1 = _silu(jnp.dot(h, gW1_ref[...], preferred_element_type=jnp.float32)
               + gb1_ref[...])
    gate = (jnp.dot(g1, gW2_ref[...], preferred_element_type=jnp.float32)
            + gb2_ref[...])[:, 0]
    batch = batch_ref[0, :]
    seg = lax.broadcasted_iota(jnp.int32, (G, N), 0)
    mask = (batch[None, :] == seg)
    neg = jnp.float32(-3e38)
    gm = jnp.max(jnp.where(mask, gate[None, :], neg), axis=1)
    gm_n = jnp.sum(jnp.where(mask, gm[:, None], 0.0), axis=0)
    gnum = jnp.exp(gate - gm_n)
    gden = jnp.sum(jnp.where(mask, gnum[None, :], 0.0), axis=1)
    gden_n = jnp.sum(jnp.where(mask, gden[:, None], 0.0), axis=0)
    a = gnum / gden_n
    wmask = jnp.where(mask, a[None, :], 0.0)
    hg = jnp.dot(wmask, h, preferred_element_type=jnp.float32)
    out_ref[...] = (jnp.dot(hg, hW_ref[...], preferred_element_type=jnp.float32)
                    + hb_ref[...])


def _pool_head(hjk, batch, params):
    return pl.pallas_call(
        _pool_body,
        out_shape=jax.ShapeDtypeStruct((G, 1), jnp.float32),
    )(hjk, batch.reshape(1, N).astype(jnp.int32),
      params['jk_W'], params['jk_b'].reshape(1, NODE_DIM),
      params['jk_g'].reshape(1, NODE_DIM), params['jk_bt'].reshape(1, NODE_DIM),
      params['g_W1'], params['g_b1'].reshape(1, NODE_DIM // 2),
      params['g_W2'], params['g_b2'].reshape(1, 1),
      params['head_W'], params['head_b'].reshape(1, 1))


def kernel(x, edge_index, edge_attr, batch, params):
    n = N
    src, dst = edge_index[0], edge_index[1]
    src3 = src.astype(jnp.int32).reshape(SC_NW, NCH, CH)
    dst3 = dst.astype(jnp.int32).reshape(SC_NW, NCH, CH)
    h = _mm_bias(x, params['atom_W'], params['atom_b'])
    e = _mm_bias(edge_attr, params['bond_W'], params['bond_b'], blk=EBLK)
    deg = _deg_sc(dst3).sum(axis=0)[:, 0]
    outs = []
    for lp in params['layers']:
        att2 = lp['att'].reshape(HEADS, HID)
        x_l, x_r = _lr_proj(h, lp['Wl'], lp['bl'], lp['Wr'], lp['br'])
        ee, w = _edge_maps(e, lp['We'], lp['eW'][2 * NODE_DIM:], lp['eb'])
        logit_e, mx_parts, sume_parts = _att_pass1_sc(
            x_l, x_r, ee, e, src3, dst3, att2)
        sum_e = sume_parts.sum(axis=0)
        logit_s = _node_mid(sum_e, deg, x_l, x_r, lp['We'], lp['att'])
        mx = jnp.maximum(mx_parts.reshape(SC_NW, n, HEADS).max(axis=0),
                         logit_s)
        num_s = jnp.exp(logit_s - mx)
        mx_pad = jnp.pad(mx, ((0, 0), (0, EDGE_DIM - HEADS)))
        part = _att_pass2_sc(logit_e, mx_pad, x_l, src3, dst3)
        aggp = part[:, :, :NODE_DIM].sum(axis=0)
        denp = part[:, :, NODE_DIM:NODE_DIM + HEADS].sum(axis=0)
        hn, u, v = _node_post(aggp, denp, x_l, num_s, lp['bias'],
                              lp['ng'], lp['nb'], lp['eW'][:NODE_DIM],
                              lp['eW'][NODE_DIM:2 * NODE_DIM])
        esum = _edge_sum_sc(u, v, w, src3, dst3)
        e = _edge_ln(esum, lp['eg'], lp['ebt'])
        h = hn
        outs.append(h)
    hjk = jnp.concatenate(outs, axis=-1)
    return _pool_head(hjk, batch, params)
